# baseline (speedup 1.0000x reference)
import jax
import jax.numpy as jnp
from jax import lax
from jax.experimental import pallas as pl

VOCAB = 8192
D = 32
ROWS = 16384
TILE = 512
NSTEPS = ROWS // TILE


def _vq_body(h_ref, emb_ref, zst_ref, idx_ref, loss_ref):
    step = pl.program_id(0)
    h = h_ref[...]
    emb = emb_ref[...]

    h2 = jnp.sum(h * h, axis=1, keepdims=True)
    e2 = jnp.sum(emb * emb, axis=1)[None, :]
    mm = lax.dot_general(h.astype(jnp.bfloat16), emb.astype(jnp.bfloat16),
                         (((1,), (1,)), ((), ())),
                         preferred_element_type=jnp.float32)
    dist = (h2 + e2) - 2.0 * mm

    minv = jnp.min(dist, axis=1, keepdims=True)
    iota = lax.broadcasted_iota(jnp.int32, (TILE, VOCAB), 1)
    idx = jnp.min(jnp.where(dist <= minv, iota, VOCAB), axis=1)
    idx_ref[...] = idx

    oh = (iota == idx[:, None]).astype(jnp.float32)
    zq = lax.dot_general(oh, emb, (((1,), (0,)), ((), ())),
                         preferred_element_type=jnp.float32,
                         precision=lax.Precision.HIGHEST)

    diff = zq - h
    zst_ref[...] = h + diff

    @pl.when(step == 0)
    def _():
        loss_ref[...] = jnp.zeros((1, 1), jnp.float32)

    loss_ref[...] = loss_ref[...] + jnp.sum(diff * diff)

    @pl.when(step == NSTEPS - 1)
    def _():
        loss_ref[...] = loss_ref[...] * (1.0 / (ROWS * D))


def kernel(h, embedding):
    B, C, H, W = h.shape
    h_flat = jnp.transpose(h, (0, 2, 3, 1)).reshape(-1, C)

    zst_flat, idx_flat, loss = pl.pallas_call(
        _vq_body,
        grid=(NSTEPS,),
        in_specs=[
            pl.BlockSpec((TILE, D), lambda i: (i, 0)),
            pl.BlockSpec((VOCAB, D), lambda i: (0, 0)),
        ],
        out_specs=[
            pl.BlockSpec((TILE, D), lambda i: (i, 0)),
            pl.BlockSpec((TILE,), lambda i: (i,)),
            pl.BlockSpec((1, 1), lambda i: (0, 0)),
        ],
        out_shape=[
            jax.ShapeDtypeStruct((ROWS, D), jnp.float32),
            jax.ShapeDtypeStruct((ROWS,), jnp.int32),
            jax.ShapeDtypeStruct((1, 1), jnp.float32),
        ],
    )(h_flat, embedding)

    z_q_st = jnp.transpose(zst_flat.reshape(B, H, W, C), (0, 3, 1, 2))
    indices = idx_flat.reshape(B, H, W)
    return (z_q_st, indices, loss[0, 0])
